# baseline (speedup 1.0000x reference)
import functools

import jax
import jax.numpy as jnp
from jax import lax
from jax.experimental import pallas as pl
from jax.experimental.pallas import tpu as pltpu
from jax.experimental.pallas import tpu_sc as plsc

N = 10000
E = 320000
D = 128

NTILES = 32
NPR = 320
NPAD = NTILES * NPR
TRASH = NPR
C = 6400
NCHUNK = E // C
G = 128
NBUF = 2
CAP = C + G
NEG_INF = float("-inf")

_mesh = plsc.VectorSubcoreMesh(core_axis_name="c", subcore_axis_name="s")
_params = pltpu.CompilerParams(needs_layout_passes=False)


def _init_acc(acc):
    ninf = jnp.full((16,), NEG_INF, dtype=jnp.float32)

    def init_row(r, _):
        for kk in range(D // 16):
            acc[r, pl.ds(kk * 16, 16)] = ninf
        return 0

    lax.fori_loop(0, NPR + 1, init_row, 0)


def _agg_chunk(n, x_hbm, sel, g_idx, rows, acc, sems):
    nsub = (n + (G - 1)) // G

    def group(gr, _):
        g0 = gr * NBUF
        with jax.named_scope("gat_issue"):
            for b in range(NBUF):
                @pl.when(g0 + b < nsub)
                def _():
                    def unp(i, _):
                        v = sel[pl.ds((g0 + b) * G + i * 16, 16)]
                        g_idx[b, pl.ds(i * 16, 16)] = lax.shift_right_logical(v, 9)
                        return 0
                    lax.fori_loop(0, G // 16, unp, 0)
                    pltpu.async_copy(x_hbm.at[g_idx.at[b]], rows.at[b], sems[b])

        for b in range(NBUF):
            @pl.when(g0 + b < nsub)
            def _():
                with jax.named_scope("gat_wait"):
                    pltpu.make_async_copy(x_hbm.at[g_idx.at[b]], rows.at[b], sems[b]).wait()
                off = (g0 + b) * G
                rem = n - off
                nq = (jnp.minimum(rem, G) + 15) // 16

                def edge16(q, _):
                    pvec = sel[pl.ds(off + q * 16, 16)]
                    dvec = pvec & 511
                    for lane in range(16):
                        dloc = dvec[lane]
                        e = q * 16 + lane
                        msg = [rows[b, e, pl.ds(kk * 16, 16)] for kk in range(D // 16)]
                        cur = [acc[dloc, pl.ds(kk * 16, 16)] for kk in range(D // 16)]
                        for kk in range(D // 16):
                            acc[dloc, pl.ds(kk * 16, 16)] = jnp.maximum(cur[kk], msg[kk])
                    return 0

                with jax.named_scope("rmw"):
                    lax.fori_loop(0, nq, edge16, 0)
        return 0

    lax.fori_loop(0, (nsub + NBUF - 1) // NBUF, group, 0)


def _sc_layer1(x_pad, src, dst):
    @functools.partial(
        pl.kernel,
        mesh=_mesh,
        compiler_params=_params,
        out_type=(
            jax.ShapeDtypeStruct((NPAD, D), jnp.float32),
            jax.ShapeDtypeStruct((NTILES, NCHUNK, CAP), jnp.int32),
            jax.ShapeDtypeStruct((NTILES, 64, 16), jnp.int32),
        ),
        scratch_types=[
            pltpu.VMEM((C,), jnp.int32),
            pltpu.VMEM((C,), jnp.int32),
            pltpu.VMEM((C,), jnp.int32),
            pltpu.VMEM((C,), jnp.int32),
            pltpu.VMEM((CAP,), jnp.int32),
            pltpu.VMEM((64, 16), jnp.int32),
            pltpu.VMEM((NBUF, G), jnp.int32),
            pltpu.VMEM((NBUF, G, D), jnp.float32),
            pltpu.VMEM((NPR + 1, D), jnp.float32),
            pltpu.SemaphoreType.DMA,
            pltpu.SemaphoreType.DMA,
            pltpu.SemaphoreType.DMA,
            pltpu.SemaphoreType.DMA,
            pltpu.SemaphoreType.DMA,
        ],
    )
    def k(x_hbm, src_hbm, dst_hbm, out_hbm, lists_hbm, counts_hbm,
          src_a, src_b, dst_a, dst_b, sel, counts_v, g_idx, rows, acc,
          sem0, sem1, semA, semB, semc):
        wid = lax.axis_index("s") * 2 + lax.axis_index("c")
        lo = wid * NPR
        sems = [sem0, sem1]

        _init_acc(acc)

        trash_vec = jnp.full((16,), TRASH, dtype=jnp.int32)

        def issue_load(c, src_v, dst_v, sem):
            pltpu.async_copy(src_hbm.at[pl.ds(c * C, C)], src_v, sem)
            pltpu.async_copy(dst_hbm.at[pl.ds(c * C, C)], dst_v, sem)

        def process(c, src_v, dst_v, sem, nsrc, ndst, nsem):
            pltpu.make_async_copy(src_hbm.at[pl.ds(c * C, C)], src_v, sem).wait()
            pltpu.make_async_copy(dst_hbm.at[pl.ds(c * C, C)], dst_v, sem).wait()

            @pl.when(c + 1 < NCHUNK)
            def _():
                issue_load(c + 1, nsrc, ndst, nsem)

            def filt(i, nvec):
                nv = nvec
                for u in range(4):
                    d = dst_v[pl.ds(i * 64 + u * 16, 16)]
                    s = src_v[pl.ds(i * 64 + u * 16, 16)]
                    dl = d - lo
                    m = (dl >= 0) & (dl < NPR)
                    mi = m.astype(jnp.int32)
                    incl = jnp.cumsum(mi)
                    pos = (nv + incl) - mi
                    packed = s * 512 + dl
                    plsc.store_scatter(sel, [pos], packed, mask=m)
                    nv = nv + plsc.all_reduce_population_count(m)
                return nv

            with jax.named_scope("filter"):
                nvec = lax.fori_loop(0, C // 64, filt, jnp.zeros((16,), jnp.int32))
            counts_v[c, pl.ds(0, 16)] = nvec
            n = nvec[0]

            for t in range(G // 16):
                sel[pl.ds(n + t * 16, 16)] = trash_vec

            spill = pltpu.async_copy(sel, lists_hbm.at[wid, c], semc)
            _agg_chunk(n, x_hbm, sel, g_idx, rows, acc, sems)
            spill.wait()

        issue_load(0, src_a, dst_a, semA)

        def pair_body(p, _):
            process(2 * p, src_a, dst_a, semA, src_b, dst_b, semB)
            process(2 * p + 1, src_b, dst_b, semB, src_a, dst_a, semA)
            return 0

        lax.fori_loop(0, NCHUNK // 2, pair_body, 0)

        pltpu.sync_copy(counts_v, counts_hbm.at[wid])
        pltpu.sync_copy(acc.at[pl.ds(0, NPR)], out_hbm.at[pl.ds(lo, NPR)])

    return k(x_pad, src, dst)


def _sc_layer2(x_pad, lists, counts):
    @functools.partial(
        pl.kernel,
        mesh=_mesh,
        compiler_params=_params,
        out_type=jax.ShapeDtypeStruct((NPAD, D), jnp.float32),
        scratch_types=[
            pltpu.VMEM((CAP,), jnp.int32),
            pltpu.VMEM((CAP,), jnp.int32),
            pltpu.VMEM((64, 16), jnp.int32),
            pltpu.VMEM((NBUF, G), jnp.int32),
            pltpu.VMEM((NBUF, G, D), jnp.float32),
            pltpu.VMEM((NPR + 1, D), jnp.float32),
            pltpu.SemaphoreType.DMA,
            pltpu.SemaphoreType.DMA,
            pltpu.SemaphoreType.DMA,
            pltpu.SemaphoreType.DMA,
        ],
    )
    def k(x_hbm, lists_hbm, counts_hbm, out_hbm,
          sel_a, sel_b, counts_v, g_idx, rows, acc,
          sem0, sem1, semA, semB):
        wid = lax.axis_index("s") * 2 + lax.axis_index("c")
        lo = wid * NPR
        sems = [sem0, sem1]

        _init_acc(acc)
        pltpu.sync_copy(counts_hbm.at[wid], counts_v)

        def issue_load(c, sel, sem):
            pltpu.async_copy(lists_hbm.at[wid, c], sel, sem)

        def process(c, sel, sem, nsel, nsem):
            pltpu.make_async_copy(lists_hbm.at[wid, c], sel, sem).wait()

            @pl.when(c + 1 < NCHUNK)
            def _():
                issue_load(c + 1, nsel, nsem)

            n = counts_v[c, pl.ds(0, 16)][0]
            _agg_chunk(n, x_hbm, sel, g_idx, rows, acc, sems)

        issue_load(0, sel_a, semA)

        def pair_body(p, _):
            process(2 * p, sel_a, semA, sel_b, semB)
            process(2 * p + 1, sel_b, semB, sel_a, semA)
            return 0

        lax.fori_loop(0, NCHUNK // 2, pair_body, 0)
        pltpu.sync_copy(acc.at[pl.ds(0, NPR)], out_hbm.at[pl.ds(lo, NPR)])

    return k(x_pad, lists, counts)


def _tc_dense(agg, x, w_l, b, w_r, relu):
    BM = 512

    def body(agg_ref, x_ref, wl_ref, b_ref, wr_ref, o_ref):
        a = agg_ref[...]
        a = jnp.where(a == NEG_INF, 0.0, a)
        acc = (
            jnp.dot(a, wl_ref[...], preferred_element_type=jnp.float32)
            + b_ref[...]
            + jnp.dot(x_ref[...], wr_ref[...], preferred_element_type=jnp.float32)
        )
        if relu:
            acc = jnp.maximum(acc, 0.0)
        o_ref[...] = acc

    return pl.pallas_call(
        body,
        grid=(NPAD // BM,),
        in_specs=[
            pl.BlockSpec((BM, D), lambda i: (i, 0)),
            pl.BlockSpec((BM, D), lambda i: (i, 0)),
            pl.BlockSpec((D, D), lambda i: (0, 0)),
            pl.BlockSpec((1, D), lambda i: (0, 0)),
            pl.BlockSpec((D, D), lambda i: (0, 0)),
        ],
        out_specs=pl.BlockSpec((BM, D), lambda i: (i, 0)),
        out_shape=jax.ShapeDtypeStruct((NPAD, D), jnp.float32),
    )(agg, x, w_l, b, w_r)


def kernel(features, edge_index, W1_l, b1, W1_r, W2_l, b2, W2_r):
    src = edge_index[0]
    dst = edge_index[1]
    x_pad = jnp.zeros((NPAD, D), jnp.float32).at[:N].set(features)

    agg1, lists, counts = _sc_layer1(x_pad, src, dst)
    h = _tc_dense(agg1, x_pad, W1_l, b1.reshape(1, D), W1_r, relu=True)

    agg2 = _sc_layer2(h, lists, counts)
    w2l = jnp.zeros((D, D), jnp.float32).at[:, :64].set(W2_l)
    w2r = jnp.zeros((D, D), jnp.float32).at[:, :64].set(W2_r)
    b2p = jnp.zeros((1, D), jnp.float32).at[0, :64].set(b2)
    out = _tc_dense(agg2, h, w2l, b2p, w2r, relu=False)
    return out[:N, :64]


# speedup vs baseline: 7.0998x; 7.0998x over previous
import functools

import jax
import jax.numpy as jnp
from jax import lax
from jax.experimental import pallas as pl
from jax.experimental.pallas import tpu as pltpu
from jax.experimental.pallas import tpu_sc as plsc

N = 10000
E = 320000
D = 128

NTILES = 32
NPR = 320
NPAD = NTILES * NPR
VHALF = NPAD // 2
TRASH = NPR
C = 6400
NCHUNK = E // C
G = 64
NBUF = 2
CAP = C + G
NEG_INF = float("-inf")

_mesh = plsc.VectorSubcoreMesh(core_axis_name="c", subcore_axis_name="s")
_params = pltpu.CompilerParams(needs_layout_passes=False)


def _sc_filter(src, dst):
    @functools.partial(
        pl.kernel,
        mesh=_mesh,
        compiler_params=_params,
        out_type=(
            jax.ShapeDtypeStruct((2, NTILES, NCHUNK, CAP), jnp.int32),
            jax.ShapeDtypeStruct((2, NTILES, 64, 16), jnp.int32),
        ),
        scratch_types=[
            pltpu.VMEM((C,), jnp.int32),
            pltpu.VMEM((C,), jnp.int32),
            pltpu.VMEM((C,), jnp.int32),
            pltpu.VMEM((C,), jnp.int32),
            pltpu.VMEM((CAP,), jnp.int32),
            pltpu.VMEM((CAP,), jnp.int32),
            pltpu.VMEM((64, 16), jnp.int32),
            pltpu.VMEM((64, 16), jnp.int32),
            pltpu.SemaphoreType.DMA,
            pltpu.SemaphoreType.DMA,
            pltpu.SemaphoreType.DMA,
        ],
    )
    def k(src_hbm, dst_hbm, lists_hbm, counts_hbm,
          src_a, src_b, dst_a, dst_b, sel_a, sel_b, cnt_a, cnt_b,
          semA, semB, semc):
        wid = lax.axis_index("s") * 2 + lax.axis_index("c")
        lo = wid * NPR
        trash_vec = jnp.full((16,), TRASH, dtype=jnp.int32)

        def issue_load(c, src_v, dst_v, sem):
            pltpu.async_copy(src_hbm.at[pl.ds(c * C, C)], src_v, sem)
            pltpu.async_copy(dst_hbm.at[pl.ds(c * C, C)], dst_v, sem)

        def process(c, src_v, dst_v, sem, nsrc, ndst, nsem):
            pltpu.make_async_copy(src_hbm.at[pl.ds(c * C, C)], src_v, sem).wait()
            pltpu.make_async_copy(dst_hbm.at[pl.ds(c * C, C)], dst_v, sem).wait()

            @pl.when(c + 1 < NCHUNK)
            def _():
                issue_load(c + 1, nsrc, ndst, nsem)

            def filt(i, carry):
                nva, nvb = carry
                for u in range(4):
                    d = dst_v[pl.ds(i * 64 + u * 16, 16)]
                    s = src_v[pl.ds(i * 64 + u * 16, 16)]
                    dl = d - lo
                    inr = (dl >= 0) & (dl < NPR)
                    half1 = s >= VHALF
                    ma = inr & (~half1)
                    mb = inr & half1
                    pa = s * 512 + dl
                    pb = pa - (VHALF * 512)
                    mia = ma.astype(jnp.int32)
                    mib = mb.astype(jnp.int32)
                    posa = (nva + jnp.cumsum(mia)) - mia
                    posb = (nvb + jnp.cumsum(mib)) - mib
                    plsc.store_scatter(sel_a, [posa], pa, mask=ma)
                    plsc.store_scatter(sel_b, [posb], pb, mask=mb)
                    nva = nva + plsc.all_reduce_population_count(ma)
                    nvb = nvb + plsc.all_reduce_population_count(mb)
                return nva, nvb

            zero = jnp.zeros((16,), jnp.int32)
            nva, nvb = lax.fori_loop(0, C // 64, filt, (zero, zero))
            cnt_a[c, pl.ds(0, 16)] = nva
            cnt_b[c, pl.ds(0, 16)] = nvb
            na = nva[0]
            nb = nvb[0]
            for t in range(G // 16):
                sel_a[pl.ds(na + t * 16, 16)] = trash_vec
                sel_b[pl.ds(nb + t * 16, 16)] = trash_vec
            pltpu.async_copy(sel_a, lists_hbm.at[0, wid, c], semc)
            spill = pltpu.async_copy(sel_b, lists_hbm.at[1, wid, c], semc)
            pltpu.make_async_copy(sel_a, lists_hbm.at[0, wid, c], semc).wait()
            spill.wait()

        issue_load(0, src_a, dst_a, semA)

        def pair_body(p, _):
            process(2 * p, src_a, dst_a, semA, src_b, dst_b, semB)
            process(2 * p + 1, src_b, dst_b, semB, src_a, dst_a, semA)
            return 0

        lax.fori_loop(0, NCHUNK // 2, pair_body, 0)

        pltpu.sync_copy(cnt_a, counts_hbm.at[0, wid])
        pltpu.sync_copy(cnt_b, counts_hbm.at[1, wid])

    return k(src, dst)


def _sc_aggregate(x_pad, lists, counts):
    @functools.partial(
        pl.kernel,
        mesh=_mesh,
        compiler_params=_params,
        out_type=jax.ShapeDtypeStruct((2, NPAD * D), jnp.float32),
        scratch_types=[
            pltpu.VMEM_SHARED((VHALF, D), jnp.float32),
            pltpu.VMEM((CAP,), jnp.int32),
            pltpu.VMEM((CAP,), jnp.int32),
            pltpu.VMEM((64, 16), jnp.int32),
            pltpu.VMEM((NBUF, G), jnp.int32),
            pltpu.VMEM((NBUF * G, D), jnp.float32),
            pltpu.VMEM(((NPR + 1) * D,), jnp.float32),
            pltpu.SemaphoreType.DMA,
            pltpu.SemaphoreType.DMA,
            pltpu.SemaphoreType.DMA,
            pltpu.SemaphoreType.DMA,
        ],
    )
    def k(x_hbm, lists_hbm, counts_hbm, out_hbm,
          table, sel_a, sel_b, counts_v, g_idx, rows, acc,
          sem0, sem1, semA, semB):
        cid = lax.axis_index("c")
        sid = lax.axis_index("s")
        sems = [sem0, sem1]

        rpt = VHALF // 16
        pltpu.sync_copy(x_hbm.at[pl.ds(cid * VHALF + sid * rpt, rpt)],
                        table.at[pl.ds(sid * rpt, rpt)])
        plsc.subcore_barrier()

        ninf = jnp.full((16,), NEG_INF, dtype=jnp.float32)

        def init_row(r, _):
            for kk in range(D // 16):
                acc[pl.ds(r * D + kk * 16, 16)] = ninf
            return 0

        def agg_chunk(n, sel):
            nsub = (n + (G - 1)) // G

            def group(gr, _):
                g0 = gr * NBUF
                for b in range(NBUF):
                    @pl.when(g0 + b < nsub)
                    def _():
                        def unp(i, _):
                            v = sel[pl.ds((g0 + b) * G + i * 16, 16)]
                            g_idx[b, pl.ds(i * 16, 16)] = lax.shift_right_logical(v, 9)
                            return 0
                        lax.fori_loop(0, G // 16, unp, 0)
                        pltpu.async_copy(table.at[g_idx.at[b]],
                                         rows.at[pl.ds(b * G, G)], sems[b])

                for b in range(NBUF):
                    @pl.when(g0 + b < nsub)
                    def _():
                        pltpu.make_async_copy(table.at[g_idx.at[b]],
                                              rows.at[pl.ds(b * G, G)], sems[b]).wait()
                        off = (g0 + b) * G
                        nq = (jnp.minimum(n - off, G) + 15) // 16

                        def edge16(q, _):
                            pvec = sel[pl.ds(off + q * 16, 16)]
                            dvec = (pvec & 511) * D
                            for lane in range(16):
                                dbase = dvec[lane]
                                e = b * G + q * 16 + lane
                                msg = [rows[e, pl.ds(kk * 16, 16)] for kk in range(D // 16)]
                                cur = [acc[pl.ds(dbase + kk * 16, 16)] for kk in range(D // 16)]
                                for kk in range(D // 16):
                                    acc[pl.ds(dbase + kk * 16, 16)] = jnp.maximum(cur[kk], msg[kk])
                            return 0

                        lax.fori_loop(0, nq, edge16, 0)
                return 0

            lax.fori_loop(0, (nsub + NBUF - 1) // NBUF, group, 0)

        for t in range(2):
            rid = sid * 2 + t
            lax.fori_loop(0, NPR + 1, init_row, 0)
            pltpu.sync_copy(counts_hbm.at[cid, rid], counts_v)

            def issue_load(c, sel, sem):
                pltpu.async_copy(lists_hbm.at[cid, rid, c], sel, sem)

            def process(c, sel, sem, nsel, nsem):
                pltpu.make_async_copy(lists_hbm.at[cid, rid, c], sel, sem).wait()

                @pl.when(c + 1 < NCHUNK)
                def _():
                    issue_load(c + 1, nsel, nsem)

                n = counts_v[c, pl.ds(0, 16)][0]
                agg_chunk(n, sel)

            issue_load(0, sel_a, semA)

            def pair_body(p, _):
                process(2 * p, sel_a, semA, sel_b, semB)
                process(2 * p + 1, sel_b, semB, sel_a, semA)
                return 0

            lax.fori_loop(0, NCHUNK // 2, pair_body, 0)
            pltpu.sync_copy(acc.at[pl.ds(0, NPR * D)],
                            out_hbm.at[cid, pl.ds(rid * NPR * D, NPR * D)])

    return k(x_pad, lists, counts)


def _tc_dense(agg2x, x, w_l, b, w_r, relu):
    BM = 512

    def body(a0_ref, a1_ref, x_ref, wl_ref, b_ref, wr_ref, o_ref):
        a = jnp.maximum(a0_ref[...], a1_ref[...])
        a = jnp.where(a == NEG_INF, 0.0, a)
        acc = (
            jnp.dot(a, wl_ref[...], preferred_element_type=jnp.float32)
            + b_ref[...]
            + jnp.dot(x_ref[...], wr_ref[...], preferred_element_type=jnp.float32)
        )
        if relu:
            acc = jnp.maximum(acc, 0.0)
        o_ref[...] = acc

    a0 = agg2x[0].reshape(NPAD, D)
    a1 = agg2x[1].reshape(NPAD, D)
    return pl.pallas_call(
        body,
        grid=(NPAD // BM,),
        in_specs=[
            pl.BlockSpec((BM, D), lambda i: (i, 0)),
            pl.BlockSpec((BM, D), lambda i: (i, 0)),
            pl.BlockSpec((BM, D), lambda i: (i, 0)),
            pl.BlockSpec((D, D), lambda i: (0, 0)),
            pl.BlockSpec((1, D), lambda i: (0, 0)),
            pl.BlockSpec((D, D), lambda i: (0, 0)),
        ],
        out_specs=pl.BlockSpec((BM, D), lambda i: (i, 0)),
        out_shape=jax.ShapeDtypeStruct((NPAD, D), jnp.float32),
    )(a0, a1, x, w_l, b, w_r)


def kernel(features, edge_index, W1_l, b1, W1_r, W2_l, b2, W2_r):
    src = edge_index[0]
    dst = edge_index[1]
    x_pad = jnp.zeros((NPAD, D), jnp.float32).at[:N].set(features)

    lists, counts = _sc_filter(src, dst)
    agg1 = _sc_aggregate(x_pad, lists, counts)
    h = _tc_dense(agg1, x_pad, W1_l, b1.reshape(1, D), W1_r, relu=True)

    agg2 = _sc_aggregate(h, lists, counts)
    w2l = jnp.zeros((D, D), jnp.float32).at[:, :64].set(W2_l)
    w2r = jnp.zeros((D, D), jnp.float32).at[:, :64].set(W2_r)
    b2p = jnp.zeros((1, D), jnp.float32).at[0, :64].set(b2)
    out = _tc_dense(agg2, h, w2l, b2p, w2r, relu=False)
    return out[:N, :64]
